# Initial kernel scaffold; baseline (speedup 1.0000x reference)
import jax
import jax.numpy as jnp
from jax.experimental import pallas as pl


def kernel(query, support, q_l1, q_deg_l, q_r1, q_deg_r, s_l1, s_deg_l, s_r1, s_deg_r, symbol_emb, gcn_w_W, gcn_w_b, gcn_b, gate_w_W, gate_w_b, gate_b, se_proj1_W, se_proj1_b, se_proj2_W, se_proj2_b, se_ln_g, se_ln_b, lstm_W_ih, lstm_W_hh, lstm_b_ih, lstm_b_hh):
    raise NotImplementedError("write your pallas kernel here")


# speedup vs baseline: 1.9071x; 1.9071x over previous
import functools

import jax
import jax.numpy as jnp
from jax import lax
from jax.experimental import pallas as pl
from jax.experimental.pallas import tpu as pltpu
from jax.experimental.pallas import tpu_sc as plsc

EMBED_DIM = 128
K_SEL = 10
KMAX = 64
D_MODEL = 256
HID = 512

_NC = 2
_NS = 16
_NW = _NC * _NS
_CHUNK = 256


def _sc_gather(table, idx, npad):
    nchunks = npad // (_NW * _CHUNK)
    mesh = plsc.VectorSubcoreMesh(core_axis_name="c", subcore_axis_name="s")

    @functools.partial(
        pl.kernel,
        mesh=mesh,
        out_type=jax.ShapeDtypeStruct((npad, EMBED_DIM), jnp.float32),
        scratch_types=[
            pltpu.VMEM((_CHUNK,), jnp.int32),
            pltpu.VMEM((_CHUNK, EMBED_DIM), jnp.float32),
            pltpu.SemaphoreType.DMA,
        ],
    )
    def gk(idx_hbm, table_hbm, out_hbm, idx_v, rows_v, sem):
        wid = lax.axis_index("s") * _NC + lax.axis_index("c")
        base0 = wid * (nchunks * _CHUNK)

        def body(i, carry):
            base = base0 + i * _CHUNK
            pltpu.sync_copy(idx_hbm.at[pl.ds(base, _CHUNK)], idx_v)
            pltpu.async_copy(table_hbm.at[idx_v], rows_v, sem).wait()
            pltpu.sync_copy(rows_v, out_hbm.at[pl.ds(base, _CHUNK)])
            return carry

        lax.fori_loop(0, nchunks, body, 0)

    return gk(idx, table)


def _ne_body(rel_ref, ent_ref, self_ref, qrel_ref, wt_ref, bv_ref, gw_ref,
             gb_ref, out_ref):
    rel = rel_ref[...]
    ent = ent_ref[...]
    se = self_ref[...]
    qr = qrel_ref[...]

    def inv_norm(x):
        return 1.0 / jnp.maximum(jnp.sqrt(jnp.sum(x * x, axis=-1)), 1e-8)

    inv_se = inv_norm(se)
    inv_qr = inv_norm(qr)
    inv_ent = inv_norm(ent)
    inv_rel = inv_norm(rel)
    dot_e = jnp.sum(ent * se[:, None, :], axis=-1)
    dot_r = jnp.sum(rel * qr[:, None, :], axis=-1)
    sim = (0.7 * dot_e * inv_se[:, None] * inv_ent
           + 0.3 * dot_r * inv_qr[:, None] * inv_rel)

    iota = lax.broadcasted_iota(jnp.int32, sim.shape, 1)
    mask = jnp.zeros_like(sim)
    simc = sim
    for _ in range(K_SEL):
        m = jnp.max(simc, axis=1, keepdims=True)
        first = jnp.min(jnp.where(simc == m, iota, KMAX), axis=1,
                        keepdims=True)
        hit = iota == first
        mask = jnp.where(hit, 1.0, mask)
        simc = jnp.where(hit, -1e30, simc)

    bb = rel.shape[0]
    rel2 = rel.reshape(bb * KMAX, EMBED_DIM)
    ent2 = ent.reshape(bb * KMAX, EMBED_DIM)
    proj = (jnp.dot(rel2, wt_ref[:EMBED_DIM, :],
                    preferred_element_type=jnp.float32)
            + jnp.dot(ent2, wt_ref[EMBED_DIM:, :],
                      preferred_element_type=jnp.float32)
            + bv_ref[...])
    proj = jnp.where(proj >= 0.0, proj, 0.01 * proj)
    proj = proj.reshape(bb, KMAX, EMBED_DIM) * mask[:, :, None]
    agg = jnp.sum(proj, axis=1) * (1.0 / (float(K_SEL) + 1e-9))
    glog = jnp.sum(agg * gw_ref[...], axis=-1) + gb_ref[0, 0]
    g = jax.nn.sigmoid(glog)[:, None]
    out_ref[...] = jnp.tanh(g * agg + (1.0 - g) * se)


def _ne_call(rel, ent, selfe, qrel, wt, bvec, gw, gbias):
    n = rel.shape[0]
    bb = 64
    grid = n // bb
    return pl.pallas_call(
        _ne_body,
        grid=(grid,),
        in_specs=[
            pl.BlockSpec((bb, KMAX, EMBED_DIM), lambda i: (i, 0, 0)),
            pl.BlockSpec((bb, KMAX, EMBED_DIM), lambda i: (i, 0, 0)),
            pl.BlockSpec((bb, EMBED_DIM), lambda i: (i, 0)),
            pl.BlockSpec((bb, EMBED_DIM), lambda i: (i, 0)),
            pl.BlockSpec((2 * EMBED_DIM, EMBED_DIM), lambda i: (0, 0)),
            pl.BlockSpec((1, EMBED_DIM), lambda i: (0, 0)),
            pl.BlockSpec((1, EMBED_DIM), lambda i: (0, 0)),
            pl.BlockSpec((1, EMBED_DIM), lambda i: (0, 0)),
        ],
        out_specs=pl.BlockSpec((bb, EMBED_DIM), lambda i: (i, 0)),
        out_shape=jax.ShapeDtypeStruct((n, EMBED_DIM), jnp.float32),
    )(rel, ent, selfe, qrel, wt, bvec, gw, gbias)


def _se_body(x_ref, w1_ref, b1_ref, w2_ref, b2_ref, g_ref, b_ref, out_ref):
    x = x_ref[...]
    h = jnp.dot(x, w1_ref[...], preferred_element_type=jnp.float32) + b1_ref[...]
    h = jnp.maximum(h, 0.0)
    h = jnp.dot(h, w2_ref[...], preferred_element_type=jnp.float32) + b2_ref[...]
    y = h + x
    mu = jnp.mean(y, axis=-1, keepdims=True)
    d = y - mu
    var = jnp.mean(d * d, axis=-1, keepdims=True)
    out_ref[...] = g_ref[...] * d / jnp.sqrt(var + 1e-5) + b_ref[...]


def _se_call(x, w1t, b1, w2t, b2, lng, lnb):
    n = x.shape[0]
    bb = 512
    return pl.pallas_call(
        _se_body,
        grid=(n // bb,),
        in_specs=[
            pl.BlockSpec((bb, D_MODEL), lambda i: (i, 0)),
            pl.BlockSpec((D_MODEL, 2 * D_MODEL), lambda i: (0, 0)),
            pl.BlockSpec((1, 2 * D_MODEL), lambda i: (0, 0)),
            pl.BlockSpec((2 * D_MODEL, D_MODEL), lambda i: (0, 0)),
            pl.BlockSpec((1, D_MODEL), lambda i: (0, 0)),
            pl.BlockSpec((1, D_MODEL), lambda i: (0, 0)),
            pl.BlockSpec((1, D_MODEL), lambda i: (0, 0)),
        ],
        out_specs=pl.BlockSpec((bb, D_MODEL), lambda i: (i, 0)),
        out_shape=jax.ShapeDtypeStruct((n, D_MODEL), jnp.float32),
    )(x, w1t, b1, w2t, b2, lng, lnb)


def _lstm_body(q_ref, sg_ref, wih_ref, whh_h_ref, whh_r_ref, bias_ref,
               out_ref):
    q = q_ref[...]
    sg = sg_ref[...]
    qw = (jnp.dot(q, wih_ref[...], preferred_element_type=jnp.float32)
          + bias_ref[...])
    rv = jnp.dot(sg, whh_r_ref[...], preferred_element_type=jnp.float32)
    c = jnp.zeros((q.shape[0], HID), jnp.float32)
    h = q
    for step in range(4):
        if step == 0:
            gates = qw
        else:
            gates = (qw + jnp.dot(h, whh_h_ref[...],
                                  preferred_element_type=jnp.float32) + rv)
        i = jax.nn.sigmoid(gates[:, :HID])
        f = jax.nn.sigmoid(gates[:, HID:2 * HID])
        g = jnp.tanh(gates[:, 2 * HID:3 * HID])
        o = jax.nn.sigmoid(gates[:, 3 * HID:])
        c = f * c + i * g
        h = q + (o * jnp.tanh(c))[:, :D_MODEL]
    out_ref[...] = jnp.sum(h * sg, axis=-1)


def _lstm_call(q, sg, wih_t, whh_h_t, whh_r_t, bias):
    n = q.shape[0]
    bb = 512
    return pl.pallas_call(
        _lstm_body,
        grid=(n // bb,),
        in_specs=[
            pl.BlockSpec((bb, D_MODEL), lambda i: (i, 0)),
            pl.BlockSpec((1, D_MODEL), lambda i: (0, 0)),
            pl.BlockSpec((D_MODEL, 4 * HID), lambda i: (0, 0)),
            pl.BlockSpec((D_MODEL, 4 * HID), lambda i: (0, 0)),
            pl.BlockSpec((D_MODEL, 4 * HID), lambda i: (0, 0)),
            pl.BlockSpec((1, 4 * HID), lambda i: (0, 0)),
        ],
        out_specs=pl.BlockSpec((bb,), lambda i: (i,)),
        out_shape=jax.ShapeDtypeStruct((n,), jnp.float32),
    )(q, sg, wih_t, whh_h_t, whh_r_t, bias)


def _pad_rows(a, n):
    return jnp.concatenate(
        [a, jnp.zeros((n - a.shape[0],) + a.shape[1:], a.dtype)], axis=0)


def kernel(query, support, q_l1, q_deg_l, q_r1, q_deg_r, s_l1, s_deg_l,
           s_r1, s_deg_r, symbol_emb, gcn_w_W, gcn_w_b, gcn_b, gate_w_W,
           gate_w_b, gate_b, se_proj1_W, se_proj1_b, se_proj2_W, se_proj2_b,
           se_ln_g, se_ln_b, lstm_W_ih, lstm_W_hh, lstm_b_ih, lstm_b_hh):
    b = query.shape[0]
    few = support.shape[0]
    nq = b + few
    nqp = ((nq + 63) // 64) * 64
    i32 = jnp.int32

    def side_neighbors(qc, sc, comp):
        arr = jnp.concatenate([qc[:, :, comp], sc[:, :, comp]], axis=0)
        return _pad_rows(arr.astype(i32), nqp).reshape(-1)

    rel_idx = jnp.concatenate(
        [side_neighbors(q_l1, s_l1, 0), side_neighbors(q_r1, s_r1, 0)])
    ent_idx = jnp.concatenate(
        [side_neighbors(q_l1, s_l1, 1), side_neighbors(q_r1, s_r1, 1)])
    self_idx = jnp.concatenate([
        _pad_rows(jnp.concatenate([query[:, 0], support[:, 0]]).astype(i32),
                  nqp),
        _pad_rows(jnp.concatenate([query[:, 1], support[:, 1]]).astype(i32),
                  nqp),
    ])
    qrel_idx = _pad_rows(
        jnp.concatenate([query[:, 2], support[:, 2]]).astype(i32), nqp)

    idx_all = jnp.concatenate([rel_idx, ent_idx, self_idx, qrel_idx])
    ntot = idx_all.shape[0]
    gran = _NW * _CHUNK
    npad = ((ntot + gran - 1) // gran) * gran
    idx_all = _pad_rows(idx_all, npad)

    rows = _sc_gather(symbol_emb, idx_all, npad)

    n2 = 2 * nqp
    nk = nqp * KMAX
    rel_rows = rows[:2 * nk].reshape(n2, KMAX, EMBED_DIM)
    ent_rows = rows[2 * nk:4 * nk].reshape(n2, KMAX, EMBED_DIM)
    self_rows = rows[4 * nk:4 * nk + n2]
    qr_rows = rows[4 * nk + n2:4 * nk + n2 + nqp]
    qrel_rows = jnp.concatenate([qr_rows, qr_rows], axis=0)

    wt = gcn_w_W.T
    bvec = (gcn_w_b + gcn_b).reshape(1, EMBED_DIM)
    gw = gate_w_W.reshape(1, EMBED_DIM)
    gbias = jnp.full((1, EMBED_DIM), gate_w_b[0] + gate_b[0], jnp.float32)

    enc = _ne_call(rel_rows, ent_rows, self_rows, qrel_rows, wt, bvec, gw,
                   gbias)

    q_left, s_left = enc[:b], enc[b:b + few]
    q_right, s_right = enc[nqp:nqp + b], enc[nqp + b:nqp + b + few]
    query_vec = jnp.concatenate([q_left, q_right], axis=-1)
    support_vec = jnp.concatenate([s_left, s_right], axis=-1)

    sep = ((nq + 511) // 512) * 512
    se_in = _pad_rows(jnp.concatenate([query_vec, support_vec], axis=0), sep)
    enc3 = _se_call(se_in, se_proj1_W.T, se_proj1_b.reshape(1, -1),
                    se_proj2_W.T, se_proj2_b.reshape(1, -1),
                    se_ln_g.reshape(1, -1), se_ln_b.reshape(1, -1))
    query_enc = enc3[:b]
    sg = jnp.mean(enc3[b:b + few], axis=0, keepdims=True)

    bias = (lstm_b_ih + lstm_b_hh).reshape(1, -1)
    scores = _lstm_call(query_enc, sg, lstm_W_ih.T,
                        lstm_W_hh[:, :D_MODEL].T, lstm_W_hh[:, D_MODEL:].T,
                        bias)
    return scores


# speedup vs baseline: 2.0124x; 1.0552x over previous
import functools

import jax
import jax.numpy as jnp
from jax import lax
from jax.experimental import pallas as pl
from jax.experimental.pallas import tpu as pltpu
from jax.experimental.pallas import tpu_sc as plsc

EMBED_DIM = 128
K_SEL = 10
KMAX = 64
D_MODEL = 256
HID = 512

_NC = 2
_NS = 16
_NW = _NC * _NS
_CHUNK = 256


def _sc_gather(table, idx, npad):
    nchunks = npad // (_NW * _CHUNK)
    assert nchunks % 2 == 0
    mesh = plsc.VectorSubcoreMesh(core_axis_name="c", subcore_axis_name="s")

    @functools.partial(
        pl.kernel,
        mesh=mesh,
        out_type=jax.ShapeDtypeStruct((npad, EMBED_DIM), jnp.float32),
        scratch_types=[
            pltpu.VMEM((nchunks * _CHUNK,), jnp.int32),
            pltpu.VMEM((_CHUNK, EMBED_DIM), jnp.float32),
            pltpu.VMEM((_CHUNK, EMBED_DIM), jnp.float32),
            pltpu.SemaphoreType.DMA,
            pltpu.SemaphoreType.DMA,
            pltpu.SemaphoreType.DMA,
            pltpu.SemaphoreType.DMA,
        ],
    )
    def gk(idx_hbm, table_hbm, out_hbm, idx_v, rows0, rows1, g0, g1, s0, s1):
        wid = lax.axis_index("s") * _NC + lax.axis_index("c")
        base0 = wid * (nchunks * _CHUNK)
        rows = (rows0, rows1)
        gsem = (g0, g1)
        ssem = (s0, s1)

        pltpu.sync_copy(idx_hbm.at[pl.ds(base0, nchunks * _CHUNK)], idx_v)

        def gstart(i, b):
            pltpu.async_copy(
                table_hbm.at[idx_v.at[pl.ds(i * _CHUNK, _CHUNK)]],
                rows[b], gsem[b])

        def sstart(i, b):
            pltpu.async_copy(
                rows[b], out_hbm.at[pl.ds(base0 + i * _CHUNK, _CHUNK)],
                ssem[b])

        def swait(b):
            pltpu.make_async_copy(
                rows[b], out_hbm.at[pl.ds(base0, _CHUNK)], ssem[b]).wait()

        def gwait(i, b):
            pltpu.make_async_copy(
                table_hbm.at[idx_v.at[pl.ds(i * _CHUNK, _CHUNK)]],
                rows[b], gsem[b]).wait()

        gstart(0, 0)

        @pl.loop(0, nchunks, step=2)
        def pair(i0):
            gwait(i0, 0)

            @pl.when(i0 > 0)
            def _():
                swait(1)

            gstart(i0 + 1, 1)
            sstart(i0, 0)
            gwait(i0 + 1, 1)
            swait(0)

            @pl.when(i0 + 2 < nchunks)
            def _():
                gstart(i0 + 2, 0)

            sstart(i0 + 1, 1)

        swait(1)

    return gk(idx, table)


def _ne_body(rel_ref, ent_ref, self_ref, qrel_ref, wt_ref, bv_ref, gw_ref,
             gb_ref, out_ref):
    rel = rel_ref[...]
    ent = ent_ref[...]
    se = self_ref[...]
    qr = qrel_ref[...]

    def inv_norm(x):
        return 1.0 / jnp.maximum(jnp.sqrt(jnp.sum(x * x, axis=-1)), 1e-8)

    inv_se = inv_norm(se)
    inv_qr = inv_norm(qr)
    inv_ent = inv_norm(ent)
    inv_rel = inv_norm(rel)
    dot_e = jnp.sum(ent * se[:, None, :], axis=-1)
    dot_r = jnp.sum(rel * qr[:, None, :], axis=-1)
    sim = (0.7 * dot_e * inv_se[:, None] * inv_ent
           + 0.3 * dot_r * inv_qr[:, None] * inv_rel)

    iota = lax.broadcasted_iota(jnp.int32, sim.shape, 1)
    mask = jnp.zeros_like(sim)
    simc = sim
    for _ in range(K_SEL):
        m = jnp.max(simc, axis=1, keepdims=True)
        first = jnp.min(jnp.where(simc == m, iota, KMAX), axis=1,
                        keepdims=True)
        hit = iota == first
        mask = jnp.where(hit, 1.0, mask)
        simc = jnp.where(hit, -1e30, simc)

    bb = rel.shape[0]
    rel2 = rel.reshape(bb * KMAX, EMBED_DIM)
    ent2 = ent.reshape(bb * KMAX, EMBED_DIM)
    proj = (jnp.dot(rel2, wt_ref[:EMBED_DIM, :],
                    preferred_element_type=jnp.float32)
            + jnp.dot(ent2, wt_ref[EMBED_DIM:, :],
                      preferred_element_type=jnp.float32)
            + bv_ref[...])
    proj = jnp.where(proj >= 0.0, proj, 0.01 * proj)
    proj = proj.reshape(bb, KMAX, EMBED_DIM) * mask[:, :, None]
    agg = jnp.sum(proj, axis=1) * (1.0 / (float(K_SEL) + 1e-9))
    glog = jnp.sum(agg * gw_ref[...], axis=-1) + gb_ref[0, 0]
    g = jax.nn.sigmoid(glog)[:, None]
    out_ref[...] = jnp.tanh(g * agg + (1.0 - g) * se)


def _ne_call(rel, ent, selfe, qrel, wt, bvec, gw, gbias):
    n = rel.shape[0]
    bb = 64
    grid = n // bb
    return pl.pallas_call(
        _ne_body,
        grid=(grid,),
        in_specs=[
            pl.BlockSpec((bb, KMAX, EMBED_DIM), lambda i: (i, 0, 0)),
            pl.BlockSpec((bb, KMAX, EMBED_DIM), lambda i: (i, 0, 0)),
            pl.BlockSpec((bb, EMBED_DIM), lambda i: (i, 0)),
            pl.BlockSpec((bb, EMBED_DIM), lambda i: (i, 0)),
            pl.BlockSpec((2 * EMBED_DIM, EMBED_DIM), lambda i: (0, 0)),
            pl.BlockSpec((1, EMBED_DIM), lambda i: (0, 0)),
            pl.BlockSpec((1, EMBED_DIM), lambda i: (0, 0)),
            pl.BlockSpec((1, EMBED_DIM), lambda i: (0, 0)),
        ],
        out_specs=pl.BlockSpec((bb, EMBED_DIM), lambda i: (i, 0)),
        out_shape=jax.ShapeDtypeStruct((n, EMBED_DIM), jnp.float32),
    )(rel, ent, selfe, qrel, wt, bvec, gw, gbias)


def _se_body(x_ref, w1_ref, b1_ref, w2_ref, b2_ref, g_ref, b_ref, out_ref):
    x = x_ref[...]
    h = jnp.dot(x, w1_ref[...], preferred_element_type=jnp.float32) + b1_ref[...]
    h = jnp.maximum(h, 0.0)
    h = jnp.dot(h, w2_ref[...], preferred_element_type=jnp.float32) + b2_ref[...]
    y = h + x
    mu = jnp.mean(y, axis=-1, keepdims=True)
    d = y - mu
    var = jnp.mean(d * d, axis=-1, keepdims=True)
    out_ref[...] = g_ref[...] * d / jnp.sqrt(var + 1e-5) + b_ref[...]


def _se_call(x, w1t, b1, w2t, b2, lng, lnb):
    n = x.shape[0]
    bb = 512
    return pl.pallas_call(
        _se_body,
        grid=(n // bb,),
        in_specs=[
            pl.BlockSpec((bb, D_MODEL), lambda i: (i, 0)),
            pl.BlockSpec((D_MODEL, 2 * D_MODEL), lambda i: (0, 0)),
            pl.BlockSpec((1, 2 * D_MODEL), lambda i: (0, 0)),
            pl.BlockSpec((2 * D_MODEL, D_MODEL), lambda i: (0, 0)),
            pl.BlockSpec((1, D_MODEL), lambda i: (0, 0)),
            pl.BlockSpec((1, D_MODEL), lambda i: (0, 0)),
            pl.BlockSpec((1, D_MODEL), lambda i: (0, 0)),
        ],
        out_specs=pl.BlockSpec((bb, D_MODEL), lambda i: (i, 0)),
        out_shape=jax.ShapeDtypeStruct((n, D_MODEL), jnp.float32),
    )(x, w1t, b1, w2t, b2, lng, lnb)


def _lstm_body(q_ref, sg_ref, wih_ref, whh_h_ref, whh_r_ref, bias_ref,
               out_ref):
    q = q_ref[...]
    sg = sg_ref[...]
    qw = (jnp.dot(q, wih_ref[...], preferred_element_type=jnp.float32)
          + bias_ref[...])
    rv = jnp.dot(sg, whh_r_ref[...], preferred_element_type=jnp.float32)
    c = jnp.zeros((q.shape[0], HID), jnp.float32)
    h = q
    for step in range(4):
        if step == 0:
            gates = qw
        else:
            gates = (qw + jnp.dot(h, whh_h_ref[...],
                                  preferred_element_type=jnp.float32) + rv)
        i = jax.nn.sigmoid(gates[:, :HID])
        f = jax.nn.sigmoid(gates[:, HID:2 * HID])
        g = jnp.tanh(gates[:, 2 * HID:3 * HID])
        o = jax.nn.sigmoid(gates[:, 3 * HID:])
        c = f * c + i * g
        h = q + (o * jnp.tanh(c))[:, :D_MODEL]
    out_ref[...] = jnp.sum(h * sg, axis=-1)


def _lstm_call(q, sg, wih_t, whh_h_t, whh_r_t, bias):
    n = q.shape[0]
    bb = 512
    return pl.pallas_call(
        _lstm_body,
        grid=(n // bb,),
        in_specs=[
            pl.BlockSpec((bb, D_MODEL), lambda i: (i, 0)),
            pl.BlockSpec((1, D_MODEL), lambda i: (0, 0)),
            pl.BlockSpec((D_MODEL, 4 * HID), lambda i: (0, 0)),
            pl.BlockSpec((D_MODEL, 4 * HID), lambda i: (0, 0)),
            pl.BlockSpec((D_MODEL, 4 * HID), lambda i: (0, 0)),
            pl.BlockSpec((1, 4 * HID), lambda i: (0, 0)),
        ],
        out_specs=pl.BlockSpec((bb,), lambda i: (i,)),
        out_shape=jax.ShapeDtypeStruct((n,), jnp.float32),
    )(q, sg, wih_t, whh_h_t, whh_r_t, bias)


def _pad_rows(a, n):
    return jnp.concatenate(
        [a, jnp.zeros((n - a.shape[0],) + a.shape[1:], a.dtype)], axis=0)


def kernel(query, support, q_l1, q_deg_l, q_r1, q_deg_r, s_l1, s_deg_l,
           s_r1, s_deg_r, symbol_emb, gcn_w_W, gcn_w_b, gcn_b, gate_w_W,
           gate_w_b, gate_b, se_proj1_W, se_proj1_b, se_proj2_W, se_proj2_b,
           se_ln_g, se_ln_b, lstm_W_ih, lstm_W_hh, lstm_b_ih, lstm_b_hh):
    b = query.shape[0]
    few = support.shape[0]
    nq = b + few
    nqp = ((nq + 63) // 64) * 64
    i32 = jnp.int32

    def side_neighbors(qc, sc, comp):
        arr = jnp.concatenate([qc[:, :, comp], sc[:, :, comp]], axis=0)
        return _pad_rows(arr.astype(i32), nqp).reshape(-1)

    rel_idx = jnp.concatenate(
        [side_neighbors(q_l1, s_l1, 0), side_neighbors(q_r1, s_r1, 0)])
    ent_idx = jnp.concatenate(
        [side_neighbors(q_l1, s_l1, 1), side_neighbors(q_r1, s_r1, 1)])
    self_idx = jnp.concatenate([
        _pad_rows(jnp.concatenate([query[:, 0], support[:, 0]]).astype(i32),
                  nqp),
        _pad_rows(jnp.concatenate([query[:, 1], support[:, 1]]).astype(i32),
                  nqp),
    ])
    qrel_idx = _pad_rows(
        jnp.concatenate([query[:, 2], support[:, 2]]).astype(i32), nqp)

    idx_all = jnp.concatenate([rel_idx, ent_idx, self_idx, qrel_idx])
    ntot = idx_all.shape[0]
    gran = _NW * _CHUNK
    npad = ((ntot + gran - 1) // gran) * gran
    idx_all = _pad_rows(idx_all, npad)

    rows = _sc_gather(symbol_emb, idx_all, npad)

    n2 = 2 * nqp
    nk = nqp * KMAX
    rel_rows = rows[:2 * nk].reshape(n2, KMAX, EMBED_DIM)
    ent_rows = rows[2 * nk:4 * nk].reshape(n2, KMAX, EMBED_DIM)
    self_rows = rows[4 * nk:4 * nk + n2]
    qr_rows = rows[4 * nk + n2:4 * nk + n2 + nqp]
    qrel_rows = jnp.concatenate([qr_rows, qr_rows], axis=0)

    wt = gcn_w_W.T
    bvec = (gcn_w_b + gcn_b).reshape(1, EMBED_DIM)
    gw = gate_w_W.reshape(1, EMBED_DIM)
    gbias = jnp.full((1, EMBED_DIM), gate_w_b[0] + gate_b[0], jnp.float32)

    enc = _ne_call(rel_rows, ent_rows, self_rows, qrel_rows, wt, bvec, gw,
                   gbias)

    q_left, s_left = enc[:b], enc[b:b + few]
    q_right, s_right = enc[nqp:nqp + b], enc[nqp + b:nqp + b + few]
    query_vec = jnp.concatenate([q_left, q_right], axis=-1)
    support_vec = jnp.concatenate([s_left, s_right], axis=-1)

    sep = ((nq + 511) // 512) * 512
    se_in = _pad_rows(jnp.concatenate([query_vec, support_vec], axis=0), sep)
    enc3 = _se_call(se_in, se_proj1_W.T, se_proj1_b.reshape(1, -1),
                    se_proj2_W.T, se_proj2_b.reshape(1, -1),
                    se_ln_g.reshape(1, -1), se_ln_b.reshape(1, -1))
    query_enc = enc3[:b]
    sg = jnp.mean(enc3[b:b + few], axis=0, keepdims=True)

    bias = (lstm_b_ih + lstm_b_hh).reshape(1, -1)
    scores = _lstm_call(query_enc, sg, lstm_W_ih.T,
                        lstm_W_hh[:, :D_MODEL].T, lstm_W_hh[:, D_MODEL:].T,
                        bias)
    return scores


# speedup vs baseline: 2.0585x; 1.0229x over previous
import functools

import jax
import jax.numpy as jnp
from jax import lax
from jax.experimental import pallas as pl
from jax.experimental.pallas import tpu as pltpu
from jax.experimental.pallas import tpu_sc as plsc

EMBED_DIM = 128
K_SEL = 10
KMAX = 64
D_MODEL = 256
HID = 512

_NC = 2
_NS = 16
_NW = _NC * _NS
_CHUNK = 128
_NBUF = 4


def _sc_gather(table, idx, npad):
    nchunks = npad // (_NW * _CHUNK)
    assert nchunks % _NBUF == 0
    mesh = plsc.VectorSubcoreMesh(core_axis_name="c", subcore_axis_name="s")

    @functools.partial(
        pl.kernel,
        mesh=mesh,
        out_type=jax.ShapeDtypeStruct((npad, EMBED_DIM), jnp.float32),
        scratch_types=(
            [pltpu.VMEM((nchunks * _CHUNK,), jnp.int32)]
            + [pltpu.VMEM((_CHUNK, EMBED_DIM), jnp.float32)] * _NBUF
            + [pltpu.SemaphoreType.DMA] * (2 * _NBUF)
        ),
    )
    def gk(idx_hbm, table_hbm, out_hbm, idx_v, *bufs):
        rows = bufs[:_NBUF]
        gsem = bufs[_NBUF:2 * _NBUF]
        ssem = bufs[2 * _NBUF:]
        wid = lax.axis_index("s") * _NC + lax.axis_index("c")
        base0 = wid * (nchunks * _CHUNK)

        pltpu.sync_copy(idx_hbm.at[pl.ds(base0, nchunks * _CHUNK)], idx_v)

        def gstart(i, b):
            pltpu.async_copy(
                table_hbm.at[idx_v.at[pl.ds(i * _CHUNK, _CHUNK)]],
                rows[b], gsem[b])

        def sstart(i, b):
            pltpu.async_copy(
                rows[b], out_hbm.at[pl.ds(base0 + i * _CHUNK, _CHUNK)],
                ssem[b])

        def swait(b):
            pltpu.make_async_copy(
                rows[b], out_hbm.at[pl.ds(base0, _CHUNK)], ssem[b]).wait()

        def gwait(i, b):
            pltpu.make_async_copy(
                table_hbm.at[idx_v.at[pl.ds(i * _CHUNK, _CHUNK)]],
                rows[b], gsem[b]).wait()

        for b in range(_NBUF):
            gstart(b, b)

        @pl.loop(0, nchunks, step=_NBUF)
        def group(i0):
            for jj in range(_NBUF):
                j = i0 + jj
                b = jj
                bprev = (jj + _NBUF - 1) % _NBUF

                gwait(j, b)

                @pl.when(j > 0)
                def _():
                    swait(bprev)

                @pl.when((j > 0) & (j + _NBUF - 1 < nchunks))
                def _():
                    gstart(j + _NBUF - 1, bprev)

                sstart(j, b)

        swait((nchunks - 1) % _NBUF)

    return gk(idx, table)


def _ne_body(rel_ref, ent_ref, self_ref, qrel_ref, wt_ref, bv_ref, gw_ref,
             gb_ref, out_ref):
    rel = rel_ref[...]
    ent = ent_ref[...]
    se = self_ref[...]
    qr = qrel_ref[...]

    def inv_norm(x):
        return 1.0 / jnp.maximum(jnp.sqrt(jnp.sum(x * x, axis=-1)), 1e-8)

    inv_se = inv_norm(se)
    inv_qr = inv_norm(qr)
    inv_ent = inv_norm(ent)
    inv_rel = inv_norm(rel)
    dot_e = jnp.sum(ent * se[:, None, :], axis=-1)
    dot_r = jnp.sum(rel * qr[:, None, :], axis=-1)
    sim = (0.7 * dot_e * inv_se[:, None] * inv_ent
           + 0.3 * dot_r * inv_qr[:, None] * inv_rel)

    iota = lax.broadcasted_iota(jnp.int32, sim.shape, 1)
    mask = jnp.zeros_like(sim)
    simc = sim
    for _ in range(K_SEL):
        m = jnp.max(simc, axis=1, keepdims=True)
        first = jnp.min(jnp.where(simc == m, iota, KMAX), axis=1,
                        keepdims=True)
        hit = iota == first
        mask = jnp.where(hit, 1.0, mask)
        simc = jnp.where(hit, -1e30, simc)

    bb = rel.shape[0]
    rel2 = rel.reshape(bb * KMAX, EMBED_DIM)
    ent2 = ent.reshape(bb * KMAX, EMBED_DIM)
    proj = (jnp.dot(rel2, wt_ref[:EMBED_DIM, :],
                    preferred_element_type=jnp.float32)
            + jnp.dot(ent2, wt_ref[EMBED_DIM:, :],
                      preferred_element_type=jnp.float32)
            + bv_ref[...])
    proj = jnp.where(proj >= 0.0, proj, 0.01 * proj)
    proj = proj.reshape(bb, KMAX, EMBED_DIM) * mask[:, :, None]
    agg = jnp.sum(proj, axis=1) * (1.0 / (float(K_SEL) + 1e-9))
    glog = jnp.sum(agg * gw_ref[...], axis=-1) + gb_ref[0, 0]
    g = jax.nn.sigmoid(glog)[:, None]
    out_ref[...] = jnp.tanh(g * agg + (1.0 - g) * se)


def _ne_call(rel, ent, selfe, qrel, wt, bvec, gw, gbias):
    n = rel.shape[0]
    bb = 64
    grid = n // bb
    return pl.pallas_call(
        _ne_body,
        grid=(grid,),
        in_specs=[
            pl.BlockSpec((bb, KMAX, EMBED_DIM), lambda i: (i, 0, 0)),
            pl.BlockSpec((bb, KMAX, EMBED_DIM), lambda i: (i, 0, 0)),
            pl.BlockSpec((bb, EMBED_DIM), lambda i: (i, 0)),
            pl.BlockSpec((bb, EMBED_DIM), lambda i: (i, 0)),
            pl.BlockSpec((2 * EMBED_DIM, EMBED_DIM), lambda i: (0, 0)),
            pl.BlockSpec((1, EMBED_DIM), lambda i: (0, 0)),
            pl.BlockSpec((1, EMBED_DIM), lambda i: (0, 0)),
            pl.BlockSpec((1, EMBED_DIM), lambda i: (0, 0)),
        ],
        out_specs=pl.BlockSpec((bb, EMBED_DIM), lambda i: (i, 0)),
        out_shape=jax.ShapeDtypeStruct((n, EMBED_DIM), jnp.float32),
    )(rel, ent, selfe, qrel, wt, bvec, gw, gbias)


def _se_body(x_ref, w1_ref, b1_ref, w2_ref, b2_ref, g_ref, b_ref, out_ref):
    x = x_ref[...]
    h = jnp.dot(x, w1_ref[...], preferred_element_type=jnp.float32) + b1_ref[...]
    h = jnp.maximum(h, 0.0)
    h = jnp.dot(h, w2_ref[...], preferred_element_type=jnp.float32) + b2_ref[...]
    y = h + x
    mu = jnp.mean(y, axis=-1, keepdims=True)
    d = y - mu
    var = jnp.mean(d * d, axis=-1, keepdims=True)
    out_ref[...] = g_ref[...] * d / jnp.sqrt(var + 1e-5) + b_ref[...]


def _se_call(x, w1t, b1, w2t, b2, lng, lnb):
    n = x.shape[0]
    bb = 512
    return pl.pallas_call(
        _se_body,
        grid=(n // bb,),
        in_specs=[
            pl.BlockSpec((bb, D_MODEL), lambda i: (i, 0)),
            pl.BlockSpec((D_MODEL, 2 * D_MODEL), lambda i: (0, 0)),
            pl.BlockSpec((1, 2 * D_MODEL), lambda i: (0, 0)),
            pl.BlockSpec((2 * D_MODEL, D_MODEL), lambda i: (0, 0)),
            pl.BlockSpec((1, D_MODEL), lambda i: (0, 0)),
            pl.BlockSpec((1, D_MODEL), lambda i: (0, 0)),
            pl.BlockSpec((1, D_MODEL), lambda i: (0, 0)),
        ],
        out_specs=pl.BlockSpec((bb, D_MODEL), lambda i: (i, 0)),
        out_shape=jax.ShapeDtypeStruct((n, D_MODEL), jnp.float32),
    )(x, w1t, b1, w2t, b2, lng, lnb)


def _lstm_body(q_ref, sg_ref, wih_ref, whh_h_ref, whh_r_ref, bias_ref,
               out_ref):
    q = q_ref[...]
    sg = sg_ref[...]
    qw = (jnp.dot(q, wih_ref[...], preferred_element_type=jnp.float32)
          + bias_ref[...])
    rv = jnp.dot(sg, whh_r_ref[...], preferred_element_type=jnp.float32)
    c = jnp.zeros((q.shape[0], HID), jnp.float32)
    h = q
    for step in range(4):
        if step == 0:
            gates = qw
        else:
            gates = (qw + jnp.dot(h, whh_h_ref[...],
                                  preferred_element_type=jnp.float32) + rv)
        i = jax.nn.sigmoid(gates[:, :HID])
        f = jax.nn.sigmoid(gates[:, HID:2 * HID])
        g = jnp.tanh(gates[:, 2 * HID:3 * HID])
        o = jax.nn.sigmoid(gates[:, 3 * HID:])
        c = f * c + i * g
        h = q + (o * jnp.tanh(c))[:, :D_MODEL]
    out_ref[...] = jnp.sum(h * sg, axis=-1)


def _lstm_call(q, sg, wih_t, whh_h_t, whh_r_t, bias):
    n = q.shape[0]
    bb = 512
    return pl.pallas_call(
        _lstm_body,
        grid=(n // bb,),
        in_specs=[
            pl.BlockSpec((bb, D_MODEL), lambda i: (i, 0)),
            pl.BlockSpec((1, D_MODEL), lambda i: (0, 0)),
            pl.BlockSpec((D_MODEL, 4 * HID), lambda i: (0, 0)),
            pl.BlockSpec((D_MODEL, 4 * HID), lambda i: (0, 0)),
            pl.BlockSpec((D_MODEL, 4 * HID), lambda i: (0, 0)),
            pl.BlockSpec((1, 4 * HID), lambda i: (0, 0)),
        ],
        out_specs=pl.BlockSpec((bb,), lambda i: (i,)),
        out_shape=jax.ShapeDtypeStruct((n,), jnp.float32),
    )(q, sg, wih_t, whh_h_t, whh_r_t, bias)


def _pad_rows(a, n):
    return jnp.concatenate(
        [a, jnp.zeros((n - a.shape[0],) + a.shape[1:], a.dtype)], axis=0)


def kernel(query, support, q_l1, q_deg_l, q_r1, q_deg_r, s_l1, s_deg_l,
           s_r1, s_deg_r, symbol_emb, gcn_w_W, gcn_w_b, gcn_b, gate_w_W,
           gate_w_b, gate_b, se_proj1_W, se_proj1_b, se_proj2_W, se_proj2_b,
           se_ln_g, se_ln_b, lstm_W_ih, lstm_W_hh, lstm_b_ih, lstm_b_hh):
    b = query.shape[0]
    few = support.shape[0]
    nq = b + few
    nqp = ((nq + 63) // 64) * 64
    i32 = jnp.int32

    def side_neighbors(qc, sc, comp):
        arr = jnp.concatenate([qc[:, :, comp], sc[:, :, comp]], axis=0)
        return _pad_rows(arr.astype(i32), nqp).reshape(-1)

    rel_idx = jnp.concatenate(
        [side_neighbors(q_l1, s_l1, 0), side_neighbors(q_r1, s_r1, 0)])
    ent_idx = jnp.concatenate(
        [side_neighbors(q_l1, s_l1, 1), side_neighbors(q_r1, s_r1, 1)])
    self_idx = jnp.concatenate([
        _pad_rows(jnp.concatenate([query[:, 0], support[:, 0]]).astype(i32),
                  nqp),
        _pad_rows(jnp.concatenate([query[:, 1], support[:, 1]]).astype(i32),
                  nqp),
    ])
    qrel_idx = _pad_rows(
        jnp.concatenate([query[:, 2], support[:, 2]]).astype(i32), nqp)

    idx_all = jnp.concatenate([rel_idx, ent_idx, self_idx, qrel_idx])
    ntot = idx_all.shape[0]
    gran = _NW * _CHUNK * _NBUF
    npad = ((ntot + gran - 1) // gran) * gran
    idx_all = _pad_rows(idx_all, npad)

    rows = _sc_gather(symbol_emb, idx_all, npad)

    n2 = 2 * nqp
    nk = nqp * KMAX
    rel_rows = rows[:2 * nk].reshape(n2, KMAX, EMBED_DIM)
    ent_rows = rows[2 * nk:4 * nk].reshape(n2, KMAX, EMBED_DIM)
    self_rows = rows[4 * nk:4 * nk + n2]
    qr_rows = rows[4 * nk + n2:4 * nk + n2 + nqp]
    qrel_rows = jnp.concatenate([qr_rows, qr_rows], axis=0)

    wt = gcn_w_W.T
    bvec = (gcn_w_b + gcn_b).reshape(1, EMBED_DIM)
    gw = gate_w_W.reshape(1, EMBED_DIM)
    gbias = jnp.full((1, EMBED_DIM), gate_w_b[0] + gate_b[0], jnp.float32)

    enc = _ne_call(rel_rows, ent_rows, self_rows, qrel_rows, wt, bvec, gw,
                   gbias)

    q_left, s_left = enc[:b], enc[b:b + few]
    q_right, s_right = enc[nqp:nqp + b], enc[nqp + b:nqp + b + few]
    query_vec = jnp.concatenate([q_left, q_right], axis=-1)
    support_vec = jnp.concatenate([s_left, s_right], axis=-1)

    sep = ((nq + 511) // 512) * 512
    se_in = _pad_rows(jnp.concatenate([query_vec, support_vec], axis=0), sep)
    enc3 = _se_call(se_in, se_proj1_W.T, se_proj1_b.reshape(1, -1),
                    se_proj2_W.T, se_proj2_b.reshape(1, -1),
                    se_ln_g.reshape(1, -1), se_ln_b.reshape(1, -1))
    query_enc = enc3[:b]
    sg = jnp.mean(enc3[b:b + few], axis=0, keepdims=True)

    bias = (lstm_b_ih + lstm_b_hh).reshape(1, -1)
    scores = _lstm_call(query_enc, sg, lstm_W_ih.T,
                        lstm_W_hh[:, :D_MODEL].T, lstm_W_hh[:, D_MODEL:].T,
                        bias)
    return scores


# speedup vs baseline: 2.2935x; 1.1142x over previous
import functools

import jax
import jax.numpy as jnp
from jax import lax
from jax.experimental import pallas as pl
from jax.experimental.pallas import tpu as pltpu
from jax.experimental.pallas import tpu_sc as plsc

EMBED_DIM = 128
K_SEL = 10
KMAX = 64
D_MODEL = 256
HID = 512

_NC = 2
_NS = 16
_NW = _NC * _NS
_CHUNK = 128
_NBUF = 4


_C0_FRAC = 0.68


def _sc_gather(table, idx, npad, seg_chunks, seg_rows):
    ct = npad // _CHUNK
    half = ct // _NS
    n0 = (int(half * _C0_FRAC) // _NBUF) * _NBUF
    n1 = half - n0
    assert n1 % _NBUF == 0 and n1 > 0
    n0t = _NS * n0
    mesh = plsc.VectorSubcoreMesh(core_axis_name="c", subcore_axis_name="s")

    @functools.partial(
        pl.kernel,
        mesh=mesh,
        out_type=tuple(
            jax.ShapeDtypeStruct((r, EMBED_DIM), jnp.float32)
            for r in seg_rows),
        scratch_types=(
            [pltpu.VMEM((n0 * _CHUNK,), jnp.int32)]
            + [pltpu.VMEM((_CHUNK, EMBED_DIM), jnp.float32)] * _NBUF
            + [pltpu.SemaphoreType.DMA] * (2 * _NBUF)
        ),
    )
    def gk(idx_hbm, table_hbm, o_rel, o_ent, o_self, o_qrel, idx_v, *bufs):
        outs = (o_rel, o_ent, o_self, o_qrel)
        rows = bufs[:_NBUF]
        gsem = bufs[_NBUF:2 * _NBUF]
        ssem = bufs[2 * _NBUF:]
        c = lax.axis_index("c")
        s = lax.axis_index("s")
        is0 = c == 0
        nch = jnp.where(is0, n0, n1)
        bc0 = jnp.where(is0, s * n0, n0t + s * n1)

        pltpu.sync_copy(idx_hbm.at[pl.ds(bc0 * _CHUNK, n1 * _CHUNK)],
                        idx_v.at[pl.ds(0, n1 * _CHUNK)])

        @pl.when(is0)
        def _():
            pltpu.sync_copy(
                idx_hbm.at[pl.ds(bc0 * _CHUNK + n1 * _CHUNK,
                                 (n0 - n1) * _CHUNK)],
                idx_v.at[pl.ds(n1 * _CHUNK, (n0 - n1) * _CHUNK)])

        def gstart(i, b):
            pltpu.async_copy(
                table_hbm.at[idx_v.at[pl.ds(i * _CHUNK, _CHUNK)]],
                rows[b], gsem[b])

        def sstart(i, b):
            g = bc0 + i
            lo = 0
            for t in range(4):
                hi = seg_chunks[t]

                @pl.when((g >= lo) & (g < hi))
                def _(t=t, lo=lo):
                    pltpu.async_copy(
                        rows[b],
                        outs[t].at[pl.ds((g - lo) * _CHUNK, _CHUNK)],
                        ssem[b])

                lo = hi

        def swait(b):
            pltpu.make_async_copy(
                rows[b], o_rel.at[pl.ds(0, _CHUNK)], ssem[b]).wait()

        def gwait(i, b):
            pltpu.make_async_copy(
                table_hbm.at[idx_v.at[pl.ds(i * _CHUNK, _CHUNK)]],
                rows[b], gsem[b]).wait()

        for b in range(_NBUF):
            gstart(b, b)

        @pl.loop(0, nch, step=_NBUF)
        def group(i0):
            for jj in range(_NBUF):
                j = i0 + jj
                b = jj
                bprev = (jj + _NBUF - 1) % _NBUF

                gwait(j, b)

                @pl.when(j > 0)
                def _():
                    swait(bprev)

                @pl.when((j > 0) & (j + _NBUF - 1 < nch))
                def _():
                    gstart(j + _NBUF - 1, bprev)

                sstart(j, b)

        swait(_NBUF - 1)

    return gk(idx, table)


def _ne_body(rel_ref, ent_ref, self_ref, qrel_ref, wt_ref, bv_ref, gw_ref,
             gb_ref, out_ref):
    rel = rel_ref[...]
    ent = ent_ref[...]
    se = self_ref[...]
    qr = qrel_ref[...]

    def inv_norm(x):
        return 1.0 / jnp.maximum(jnp.sqrt(jnp.sum(x * x, axis=-1)), 1e-8)

    inv_se = inv_norm(se)
    inv_qr = inv_norm(qr)
    inv_ent = inv_norm(ent)
    inv_rel = inv_norm(rel)
    dot_e = jnp.sum(ent * se[:, None, :], axis=-1)
    dot_r = jnp.sum(rel * qr[:, None, :], axis=-1)
    sim = (0.7 * dot_e * inv_se[:, None] * inv_ent
           + 0.3 * dot_r * inv_qr[:, None] * inv_rel)

    iota = lax.broadcasted_iota(jnp.int32, sim.shape, 1)
    mask = jnp.zeros_like(sim)
    simc = sim
    for _ in range(K_SEL):
        m = jnp.max(simc, axis=1, keepdims=True)
        first = jnp.min(jnp.where(simc == m, iota, KMAX), axis=1,
                        keepdims=True)
        hit = iota == first
        mask = jnp.where(hit, 1.0, mask)
        simc = jnp.where(hit, -1e30, simc)

    bb = rel.shape[0]
    rel2 = rel.reshape(bb * KMAX, EMBED_DIM)
    ent2 = ent.reshape(bb * KMAX, EMBED_DIM)
    proj = (jnp.dot(rel2, wt_ref[:EMBED_DIM, :],
                    preferred_element_type=jnp.float32)
            + jnp.dot(ent2, wt_ref[EMBED_DIM:, :],
                      preferred_element_type=jnp.float32)
            + bv_ref[...])
    proj = jnp.where(proj >= 0.0, proj, 0.01 * proj)
    proj = proj.reshape(bb, KMAX, EMBED_DIM) * mask[:, :, None]
    agg = jnp.sum(proj, axis=1) * (1.0 / (float(K_SEL) + 1e-9))
    glog = jnp.sum(agg * gw_ref[...], axis=-1) + gb_ref[0, 0]
    g = jax.nn.sigmoid(glog)[:, None]
    out_ref[...] = jnp.tanh(g * agg + (1.0 - g) * se)


def _ne_call(rel, ent, selfe, qrel, wt, bvec, gw, gbias):
    n = rel.shape[0]
    bb = 64
    grid = n // bb
    return pl.pallas_call(
        _ne_body,
        grid=(grid,),
        in_specs=[
            pl.BlockSpec((bb, KMAX, EMBED_DIM), lambda i: (i, 0, 0)),
            pl.BlockSpec((bb, KMAX, EMBED_DIM), lambda i: (i, 0, 0)),
            pl.BlockSpec((bb, EMBED_DIM), lambda i: (i, 0)),
            pl.BlockSpec((bb, EMBED_DIM), lambda i: (i, 0)),
            pl.BlockSpec((2 * EMBED_DIM, EMBED_DIM), lambda i: (0, 0)),
            pl.BlockSpec((1, EMBED_DIM), lambda i: (0, 0)),
            pl.BlockSpec((1, EMBED_DIM), lambda i: (0, 0)),
            pl.BlockSpec((1, EMBED_DIM), lambda i: (0, 0)),
        ],
        out_specs=pl.BlockSpec((bb, EMBED_DIM), lambda i: (i, 0)),
        out_shape=jax.ShapeDtypeStruct((n, EMBED_DIM), jnp.float32),
    )(rel, ent, selfe, qrel, wt, bvec, gw, gbias)


def _se_body(x_ref, w1_ref, b1_ref, w2_ref, b2_ref, g_ref, b_ref, out_ref):
    x = x_ref[...]
    h = jnp.dot(x, w1_ref[...], preferred_element_type=jnp.float32) + b1_ref[...]
    h = jnp.maximum(h, 0.0)
    h = jnp.dot(h, w2_ref[...], preferred_element_type=jnp.float32) + b2_ref[...]
    y = h + x
    mu = jnp.mean(y, axis=-1, keepdims=True)
    d = y - mu
    var = jnp.mean(d * d, axis=-1, keepdims=True)
    out_ref[...] = g_ref[...] * d / jnp.sqrt(var + 1e-5) + b_ref[...]


def _se_call(x, w1t, b1, w2t, b2, lng, lnb):
    n = x.shape[0]
    bb = 512
    return pl.pallas_call(
        _se_body,
        grid=(n // bb,),
        in_specs=[
            pl.BlockSpec((bb, D_MODEL), lambda i: (i, 0)),
            pl.BlockSpec((D_MODEL, 2 * D_MODEL), lambda i: (0, 0)),
            pl.BlockSpec((1, 2 * D_MODEL), lambda i: (0, 0)),
            pl.BlockSpec((2 * D_MODEL, D_MODEL), lambda i: (0, 0)),
            pl.BlockSpec((1, D_MODEL), lambda i: (0, 0)),
            pl.BlockSpec((1, D_MODEL), lambda i: (0, 0)),
            pl.BlockSpec((1, D_MODEL), lambda i: (0, 0)),
        ],
        out_specs=pl.BlockSpec((bb, D_MODEL), lambda i: (i, 0)),
        out_shape=jax.ShapeDtypeStruct((n, D_MODEL), jnp.float32),
    )(x, w1t, b1, w2t, b2, lng, lnb)


def _lstm_body(q_ref, sg_ref, wih_ref, whh_h_ref, whh_r_ref, bias_ref,
               out_ref):
    q = q_ref[...]
    sg = sg_ref[...]
    qw = (jnp.dot(q, wih_ref[...], preferred_element_type=jnp.float32)
          + bias_ref[...])
    rv = jnp.dot(sg, whh_r_ref[...], preferred_element_type=jnp.float32)
    c = jnp.zeros((q.shape[0], HID), jnp.float32)
    h = q
    for step in range(4):
        if step == 0:
            gates = qw
        else:
            gates = (qw + jnp.dot(h, whh_h_ref[...],
                                  preferred_element_type=jnp.float32) + rv)
        i = jax.nn.sigmoid(gates[:, :HID])
        f = jax.nn.sigmoid(gates[:, HID:2 * HID])
        g = jnp.tanh(gates[:, 2 * HID:3 * HID])
        o = jax.nn.sigmoid(gates[:, 3 * HID:])
        c = f * c + i * g
        h = q + (o * jnp.tanh(c))[:, :D_MODEL]
    out_ref[...] = jnp.sum(h * sg, axis=-1)


def _lstm_call(q, sg, wih_t, whh_h_t, whh_r_t, bias):
    n = q.shape[0]
    bb = 512
    return pl.pallas_call(
        _lstm_body,
        grid=(n // bb,),
        in_specs=[
            pl.BlockSpec((bb, D_MODEL), lambda i: (i, 0)),
            pl.BlockSpec((1, D_MODEL), lambda i: (0, 0)),
            pl.BlockSpec((D_MODEL, 4 * HID), lambda i: (0, 0)),
            pl.BlockSpec((D_MODEL, 4 * HID), lambda i: (0, 0)),
            pl.BlockSpec((D_MODEL, 4 * HID), lambda i: (0, 0)),
            pl.BlockSpec((1, 4 * HID), lambda i: (0, 0)),
        ],
        out_specs=pl.BlockSpec((bb,), lambda i: (i,)),
        out_shape=jax.ShapeDtypeStruct((n,), jnp.float32),
    )(q, sg, wih_t, whh_h_t, whh_r_t, bias)


def _pad_rows(a, n):
    return jnp.concatenate(
        [a, jnp.zeros((n - a.shape[0],) + a.shape[1:], a.dtype)], axis=0)


def kernel(query, support, q_l1, q_deg_l, q_r1, q_deg_r, s_l1, s_deg_l,
           s_r1, s_deg_r, symbol_emb, gcn_w_W, gcn_w_b, gcn_b, gate_w_W,
           gate_w_b, gate_b, se_proj1_W, se_proj1_b, se_proj2_W, se_proj2_b,
           se_ln_g, se_ln_b, lstm_W_ih, lstm_W_hh, lstm_b_ih, lstm_b_hh):
    b = query.shape[0]
    few = support.shape[0]
    nq = b + few
    nqp = ((nq + 63) // 64) * 64
    i32 = jnp.int32

    def side_neighbors(qc, sc, comp):
        arr = jnp.concatenate([qc[:, :, comp], sc[:, :, comp]], axis=0)
        return _pad_rows(arr.astype(i32), nqp).reshape(-1)

    rel_idx = jnp.concatenate(
        [side_neighbors(q_l1, s_l1, 0), side_neighbors(q_r1, s_r1, 0)])
    ent_idx = jnp.concatenate(
        [side_neighbors(q_l1, s_l1, 1), side_neighbors(q_r1, s_r1, 1)])
    self_idx = jnp.concatenate([
        _pad_rows(jnp.concatenate([query[:, 0], support[:, 0]]).astype(i32),
                  nqp),
        _pad_rows(jnp.concatenate([query[:, 1], support[:, 1]]).astype(i32),
                  nqp),
    ])
    qrel_idx = _pad_rows(
        jnp.concatenate([query[:, 2], support[:, 2]]).astype(i32), nqp)

    idx_all = jnp.concatenate([rel_idx, ent_idx, self_idx, qrel_idx])
    ntot = idx_all.shape[0]
    gran = _NW * _CHUNK * _NBUF
    npad = ((ntot + gran - 1) // gran) * gran
    idx_all = _pad_rows(idx_all, npad)

    n2 = 2 * nqp
    nk = nqp * KMAX
    seg_rows = (2 * nk, 2 * nk, n2, npad - 4 * nk - n2)
    cum, seg_chunks = 0, []
    for r in seg_rows:
        assert r % _CHUNK == 0
        cum += r // _CHUNK
        seg_chunks.append(cum)

    o_rel, o_ent, o_self, o_qrel = _sc_gather(
        symbol_emb, idx_all, npad, tuple(seg_chunks), seg_rows)

    rel_rows = o_rel.reshape(n2, KMAX, EMBED_DIM)
    ent_rows = o_ent.reshape(n2, KMAX, EMBED_DIM)
    self_rows = o_self
    qrel_rows = jnp.concatenate([o_qrel[:nqp], o_qrel[:nqp]], axis=0)

    wt = gcn_w_W.T
    bvec = (gcn_w_b + gcn_b).reshape(1, EMBED_DIM)
    gw = gate_w_W.reshape(1, EMBED_DIM)
    gbias = jnp.full((1, EMBED_DIM), gate_w_b[0] + gate_b[0], jnp.float32)

    enc = _ne_call(rel_rows, ent_rows, self_rows, qrel_rows, wt, bvec, gw,
                   gbias)

    q_left, s_left = enc[:b], enc[b:b + few]
    q_right, s_right = enc[nqp:nqp + b], enc[nqp + b:nqp + b + few]
    query_vec = jnp.concatenate([q_left, q_right], axis=-1)
    support_vec = jnp.concatenate([s_left, s_right], axis=-1)

    sep = ((nq + 511) // 512) * 512
    se_in = _pad_rows(jnp.concatenate([query_vec, support_vec], axis=0), sep)
    enc3 = _se_call(se_in, se_proj1_W.T, se_proj1_b.reshape(1, -1),
                    se_proj2_W.T, se_proj2_b.reshape(1, -1),
                    se_ln_g.reshape(1, -1), se_ln_b.reshape(1, -1))
    query_enc = enc3[:b]
    sg = jnp.mean(enc3[b:b + few], axis=0, keepdims=True)

    bias = (lstm_b_ih + lstm_b_hh).reshape(1, -1)
    scores = _lstm_call(query_enc, sg, lstm_W_ih.T,
                        lstm_W_hh[:, :D_MODEL].T, lstm_W_hh[:, D_MODEL:].T,
                        bias)
    return scores
